# baseline (speedup 1.0000x reference)
import functools

import jax
import jax.numpy as jnp
from jax import lax
from jax.experimental import pallas as pl
from jax.experimental.pallas import tpu as pltpu
from jax.experimental.pallas import tpu_sc as plsc

_NC = 2
_NS = 16
_NW = _NC * _NS
_LANE = 128


def _iota16():
    return lax.iota(jnp.int32, 16)


def _splat16(v):
    return jnp.full((16,), v, jnp.int32)


@functools.lru_cache(maxsize=None)
def _make_detile(V, D):
    rows_out = V * D // _LANE
    nodes_per_chunk = 2 * _LANE
    rows_per_chunk = nodes_per_chunk * D // _LANE
    n_chunks = V // nodes_per_chunk
    tail_nodes = V - n_chunks * nodes_per_chunk
    base_cnt, extra = divmod(n_chunks, _NW)
    mesh = plsc.VectorSubcoreMesh(core_axis_name="c", subcore_axis_name="s")

    @functools.partial(
        pl.kernel,
        out_type=jax.ShapeDtypeStruct((rows_out, _LANE), jnp.float32),
        mesh=mesh,
        scratch_types=[
            pltpu.VMEM((nodes_per_chunk, D), jnp.float32),
            pltpu.VMEM((nodes_per_chunk, D), jnp.float32),
            pltpu.VMEM((rows_per_chunk, _LANE), jnp.float32),
            pltpu.VMEM((rows_per_chunk, _LANE), jnp.float32),
            pltpu.SemaphoreType.DMA,
            pltpu.SemaphoreType.DMA,
            pltpu.SemaphoreType.DMA,
            pltpu.SemaphoreType.DMA,
        ],
        compiler_params=pltpu.CompilerParams(use_tc_tiling_on_sc=True),
    )
    def detile_kernel(table_p, table_lin, in_a, in_b, out_a, out_b,
                      rsem_a, rsem_b, wsem_a, wsem_b):
        ins = (in_a, in_b)
        outs = (out_a, out_b)
        rsems = (rsem_a, rsem_b)
        wsems = (wsem_a, wsem_b)
        wid = lax.axis_index("s") * _NC + lax.axis_index("c")
        start = wid * base_cnt + jnp.minimum(wid, extra)
        count = base_cnt + jnp.where(wid < extra, 1, 0)

        def fire_read(slot, c):
            pltpu.async_copy(
                table_p.at[pl.ds(c * nodes_per_chunk, nodes_per_chunk)],
                ins[slot], rsems[slot],
            )

        def wait_read(slot):
            pltpu.make_async_copy(
                table_p.at[pl.ds(0, nodes_per_chunk)], ins[slot], rsems[slot]
            ).wait()

        def fire_write(slot, c):
            pltpu.async_copy(
                outs[slot],
                table_lin.at[pl.ds(c * rows_per_chunk, rows_per_chunk)],
                wsems[slot],
            )

        def wait_write(slot):
            pltpu.make_async_copy(
                outs[slot], table_lin.at[pl.ds(0, rows_per_chunk)], wsems[slot]
            ).wait()

        def repack(slot, n_rows):
            src, dst = ins[slot], outs[slot]

            @plsc.parallel_loop(0, n_rows, unroll=2)
            def _(r):
                for h in range(_LANE // 16):
                    v = src[4 * r + h // 2, pl.ds((h % 2) * 16, 16)]
                    dst[r, pl.ds(16 * h, 16)] = v

        fire_read(0, start)

        def step(t2, carry):
            for b in range(2):
                i = 2 * t2 + b

                @pl.when(i < count)
                def _():
                    @pl.when(i + 1 < count)
                    def _():
                        fire_read(1 - b, start + i + 1)

                    wait_read(b)

                    @pl.when(i >= 2)
                    def _():
                        wait_write(b)

                    repack(b, rows_per_chunk)
                    fire_write(b, start + i)
            return carry

        lax.fori_loop(0, (base_cnt + 2) // 2, step, 0)
        wait_write(0)

        @pl.when(count >= 2)
        def _():
            wait_write(1)

        @pl.when(wid == _NW - 1)
        def _():
            n_tail_rows = tail_nodes * D // _LANE
            pltpu.sync_copy(
                table_p.at[pl.ds(n_chunks * nodes_per_chunk, tail_nodes)],
                in_a.at[pl.ds(0, tail_nodes)],
            )
            repack(0, n_tail_rows)
            pltpu.sync_copy(
                out_a.at[pl.ds(0, n_tail_rows)],
                table_lin.at[pl.ds(n_chunks * rows_per_chunk, n_tail_rows)],
            )

    return detile_kernel


@functools.lru_cache(maxsize=None)
def _make_gather(B, V, D):
    n_blocks = B // _LANE
    blocks_per_w = n_blocks // _NW
    n_bands = D // 8
    mesh = plsc.VectorSubcoreMesh(core_axis_name="c", subcore_axis_name="s")

    @functools.partial(
        pl.kernel,
        out_type=jax.ShapeDtypeStruct((n_bands, n_blocks, 8, _LANE), jnp.float32),
        mesh=mesh,
        scratch_types=[
            pltpu.VMEM((blocks_per_w, _LANE), jnp.int32),
            pltpu.VMEM((_LANE, D), jnp.float32),
            pltpu.VMEM((_LANE, D), jnp.float32),
            pltpu.VMEM((_LANE, D), jnp.float32),
            pltpu.VMEM((_LANE, D), jnp.float32),
            pltpu.VMEM((D, _LANE + 1), jnp.float32),
            pltpu.VMEM((D, _LANE + 1), jnp.float32),
            pltpu.SemaphoreType.DMA,
            pltpu.SemaphoreType.DMA,
            pltpu.SemaphoreType.DMA,
            pltpu.SemaphoreType.DMA,
            pltpu.SemaphoreType.DMA,
            pltpu.SemaphoreType.DMA,
        ],
        compiler_params=pltpu.CompilerParams(
            use_tc_tiling_on_sc=False, needs_layout_passes=False
        ),
    )
    def gather_kernel(idx_hbm, table_hbm, out4, idx_v,
                      rows_a, rows_b, rows_c, rows_d,
                      band_a, band_b,
                      gsem_a, gsem_b, gsem_c, gsem_d, osem_a, osem_b):
        rows = (rows_a, rows_b, rows_c, rows_d)
        bands = (band_a, band_b)
        gsems = (gsem_a, gsem_b, gsem_c, gsem_d)
        osems = (osem_a, osem_b)
        wid = lax.axis_index("s") * _NC + lax.axis_index("c")
        base = wid * blocks_per_w
        pltpu.sync_copy(idx_hbm.at[pl.ds(base, blocks_per_w)], idx_v)

        def fire(slot, i):
            pltpu.async_copy(table_hbm.at[idx_v.at[i]], rows[slot], gsems[slot])

        def wait_gather(slot):
            pltpu.make_async_copy(
                table_hbm.at[pl.ds(0, _LANE)], rows[slot], gsems[slot]
            ).wait()

        def write_bands(slot, nt):
            for g in range(n_bands):
                pltpu.async_copy(
                    bands[slot].at[pl.ds(8 * g, 8), pl.ds(0, _LANE)],
                    out4.at[g, nt],
                    osems[slot],
                )

        def wait_bands(slot):
            for g in range(n_bands):
                pltpu.make_async_copy(
                    bands[slot].at[pl.ds(8 * g, 8), pl.ds(0, _LANE)],
                    out4.at[g, 0],
                    osems[slot],
                ).wait()

        iotas = [16 * h + _iota16() for h in range(D // 16)]

        def transpose(rslot, bslot):
            src, dst = rows[rslot], bands[bslot]

            @plsc.parallel_loop(0, _LANE, unroll=4)
            def _(l):
                sl = _splat16(l)
                for h in range(D // 16):
                    v = src[l, pl.ds(16 * h, 16)]
                    plsc.store_scatter(dst, [iotas[h], sl], v)

        for s in range(3):
            fire(s, s)

        def step(t4, carry):
            for b in range(4):
                i = 4 * t4 + b

                @pl.when(i + 3 < blocks_per_w)
                def _():
                    fire((b + 3) % 4, i + 3)

                wait_gather(b)

                @pl.when(i >= 2)
                def _():
                    wait_bands(b % 2)

                transpose(b, b % 2)
                write_bands(b % 2, base + i)
            return carry

        lax.fori_loop(0, blocks_per_w // 4, step, 0)
        wait_bands(0)
        wait_bands(1)

    return gather_kernel


def kernel(walks, table):
    (B,) = walks.shape
    V, D = table.shape
    table_lin = _make_detile(V, D)(table)
    table_rows = table_lin.reshape(V, D)
    idx2d = walks.astype(jnp.int32).reshape(B // _LANE, _LANE)
    out4 = _make_gather(B, V, D)(idx2d, table_rows)
    return out4.transpose(1, 3, 0, 2).reshape(B, D)


# speedup vs baseline: 1.0039x; 1.0039x over previous
import functools

import jax
import jax.numpy as jnp
from jax import lax
from jax.experimental import layout as jlayout
from jax.experimental import pallas as pl
from jax.experimental.pallas import tpu as pltpu
from jax.experimental.pallas import tpu_sc as plsc

_NC = 2
_NS = 16
_NW = _NC * _NS
_LANE = 128


def _iota16():
    return lax.iota(jnp.int32, 16)


def _splat16(v):
    return jnp.full((16,), v, jnp.int32)


@functools.lru_cache(maxsize=None)
def _make_detile(V, D):
    rows_out = V * D // _LANE
    nodes_per_chunk = 2 * _LANE
    rows_per_chunk = nodes_per_chunk * D // _LANE
    n_chunks = V // nodes_per_chunk
    tail_nodes = V - n_chunks * nodes_per_chunk
    base_cnt, extra = divmod(n_chunks, _NW)
    mesh = plsc.VectorSubcoreMesh(core_axis_name="c", subcore_axis_name="s")

    @functools.partial(
        pl.kernel,
        out_type=jax.ShapeDtypeStruct((rows_out, _LANE), jnp.float32),
        mesh=mesh,
        scratch_types=[
            pltpu.VMEM((nodes_per_chunk, D), jnp.float32),
            pltpu.VMEM((nodes_per_chunk, D), jnp.float32),
            pltpu.VMEM((nodes_per_chunk, D), jnp.float32),
            pltpu.VMEM((rows_per_chunk, _LANE), jnp.float32),
            pltpu.VMEM((rows_per_chunk, _LANE), jnp.float32),
            pltpu.VMEM((rows_per_chunk, _LANE), jnp.float32),
            pltpu.SemaphoreType.DMA,
            pltpu.SemaphoreType.DMA,
            pltpu.SemaphoreType.DMA,
            pltpu.SemaphoreType.DMA,
            pltpu.SemaphoreType.DMA,
            pltpu.SemaphoreType.DMA,
        ],
        compiler_params=pltpu.CompilerParams(use_tc_tiling_on_sc=True),
    )
    def detile_kernel(table_p, table_lin, in_a, in_b, in_c,
                      out_a, out_b, out_c,
                      rsem_a, rsem_b, rsem_c, wsem_a, wsem_b, wsem_c):
        ins = (in_a, in_b, in_c)
        outs = (out_a, out_b, out_c)
        rsems = (rsem_a, rsem_b, rsem_c)
        wsems = (wsem_a, wsem_b, wsem_c)
        wid = lax.axis_index("s") * _NC + lax.axis_index("c")
        start = wid * base_cnt + jnp.minimum(wid, extra)
        count = base_cnt + jnp.where(wid < extra, 1, 0)

        def fire_read(slot, c):
            pltpu.async_copy(
                table_p.at[pl.ds(c * nodes_per_chunk, nodes_per_chunk)],
                ins[slot], rsems[slot],
            )

        def wait_read(slot):
            pltpu.make_async_copy(
                table_p.at[pl.ds(0, nodes_per_chunk)], ins[slot], rsems[slot]
            ).wait()

        def fire_write(slot, c):
            pltpu.async_copy(
                outs[slot],
                table_lin.at[pl.ds(c * rows_per_chunk, rows_per_chunk)],
                wsems[slot],
            )

        def wait_write(slot):
            pltpu.make_async_copy(
                outs[slot], table_lin.at[pl.ds(0, rows_per_chunk)], wsems[slot]
            ).wait()

        def repack(slot, n_rows):
            src, dst = ins[slot], outs[slot]

            @plsc.parallel_loop(0, n_rows, unroll=2)
            def _(r):
                for h in range(_LANE // 16):
                    v = src[4 * r + h // 2, pl.ds((h % 2) * 16, 16)]
                    dst[r, pl.ds(16 * h, 16)] = v

        fire_read(0, start)
        fire_read(1, start + 1)

        def step(t3, carry):
            for b in range(3):
                i = 3 * t3 + b

                @pl.when(i < count)
                def _():
                    @pl.when(i + 2 < count)
                    def _():
                        fire_read((b + 2) % 3, start + i + 2)

                    wait_read(b)

                    @pl.when(i >= 3)
                    def _():
                        wait_write(b)

                    repack(b, rows_per_chunk)
                    fire_write(b, start + i)
            return carry

        lax.fori_loop(0, (base_cnt + 3) // 3, step, 0)
        wait_write(0)
        wait_write(1)
        wait_write(2)

        @pl.when(wid == _NW - 1)
        def _():
            n_tail_rows = tail_nodes * D // _LANE
            pltpu.sync_copy(
                table_p.at[pl.ds(n_chunks * nodes_per_chunk, tail_nodes)],
                in_a.at[pl.ds(0, tail_nodes)],
            )
            repack(0, n_tail_rows)
            pltpu.sync_copy(
                out_a.at[pl.ds(0, n_tail_rows)],
                table_lin.at[pl.ds(n_chunks * rows_per_chunk, n_tail_rows)],
            )

    return detile_kernel


@functools.lru_cache(maxsize=None)
def _make_gather(B, V, D):
    n_blocks = B // _LANE
    blocks_per_w = n_blocks // _NW
    n_bands = D // 8
    mesh = plsc.VectorSubcoreMesh(core_axis_name="c", subcore_axis_name="s")

    @functools.partial(
        pl.kernel,
        out_type=jax.ShapeDtypeStruct((n_bands, n_blocks, 8, _LANE), jnp.float32),
        mesh=mesh,
        scratch_types=[
            pltpu.VMEM((blocks_per_w, _LANE), jnp.int32),
            pltpu.VMEM((_LANE, D), jnp.float32),
            pltpu.VMEM((_LANE, D), jnp.float32),
            pltpu.VMEM((_LANE, D), jnp.float32),
            pltpu.VMEM((_LANE, D), jnp.float32),
            pltpu.VMEM((D, _LANE + 1), jnp.float32),
            pltpu.VMEM((D, _LANE + 1), jnp.float32),
            pltpu.SemaphoreType.DMA,
            pltpu.SemaphoreType.DMA,
            pltpu.SemaphoreType.DMA,
            pltpu.SemaphoreType.DMA,
            pltpu.SemaphoreType.DMA,
            pltpu.SemaphoreType.DMA,
        ],
        compiler_params=pltpu.CompilerParams(
            use_tc_tiling_on_sc=False, needs_layout_passes=False
        ),
    )
    def gather_kernel(idx_hbm, table_hbm, out4, idx_v,
                      rows_a, rows_b, rows_c, rows_d,
                      band_a, band_b,
                      gsem_a, gsem_b, gsem_c, gsem_d, osem_a, osem_b):
        rows = (rows_a, rows_b, rows_c, rows_d)
        bands = (band_a, band_b)
        gsems = (gsem_a, gsem_b, gsem_c, gsem_d)
        osems = (osem_a, osem_b)
        wid = lax.axis_index("s") * _NC + lax.axis_index("c")
        base = wid * blocks_per_w
        pltpu.sync_copy(idx_hbm.at[pl.ds(base, blocks_per_w)], idx_v)

        def fire(slot, i):
            pltpu.async_copy(table_hbm.at[idx_v.at[i]], rows[slot], gsems[slot])

        def wait_gather(slot):
            pltpu.make_async_copy(
                table_hbm.at[pl.ds(0, _LANE)], rows[slot], gsems[slot]
            ).wait()

        def write_bands(slot, nt):
            for g in range(n_bands):
                pltpu.async_copy(
                    bands[slot].at[pl.ds(8 * g, 8), pl.ds(0, _LANE)],
                    out4.at[g, nt],
                    osems[slot],
                )

        def wait_bands(slot):
            for g in range(n_bands):
                pltpu.make_async_copy(
                    bands[slot].at[pl.ds(8 * g, 8), pl.ds(0, _LANE)],
                    out4.at[g, 0],
                    osems[slot],
                ).wait()

        iotas = [16 * h + _iota16() for h in range(D // 16)]

        def transpose(rslot, bslot):
            src, dst = rows[rslot], bands[bslot]

            @plsc.parallel_loop(0, _LANE, unroll=4)
            def _(l):
                sl = _splat16(l)
                for h in range(D // 16):
                    v = src[l, pl.ds(16 * h, 16)]
                    plsc.store_scatter(dst, [iotas[h], sl], v)

        for s in range(3):
            fire(s, s)

        def step(t4, carry):
            for b in range(4):
                i = 4 * t4 + b

                @pl.when(i + 3 < blocks_per_w)
                def _():
                    fire((b + 3) % 4, i + 3)

                wait_gather(b)

                @pl.when(i >= 2)
                def _():
                    wait_bands(b % 2)

                transpose(b, b % 2)
                write_bands(b % 2, base + i)
            return carry

        lax.fori_loop(0, blocks_per_w // 4, step, 0)
        wait_bands(0)
        wait_bands(1)

    return gather_kernel


def kernel(walks, table):
    (B,) = walks.shape
    V, D = table.shape
    table_lin = _make_detile(V, D)(table)
    table_rows = table_lin.reshape(V, D)
    idx2d = walks.astype(jnp.int32).reshape(B // _LANE, _LANE)
    out4 = _make_gather(B, V, D)(idx2d, table_rows)
    return out4.transpose(1, 3, 0, 2).reshape(B, D)


# speedup vs baseline: 1.0370x; 1.0330x over previous
import functools

import jax
import jax.numpy as jnp
from jax import lax
from jax.experimental import pallas as pl
from jax.experimental.pallas import tpu as pltpu
from jax.experimental.pallas import tpu_sc as plsc

_NC = 2
_NS = 16
_NW = _NC * _NS
_LANE = 128


def _iota16():
    return lax.iota(jnp.int32, 16)


def _splat16(v):
    return jnp.full((16,), v, jnp.int32)


@functools.lru_cache(maxsize=None)
def _make_gather(B, V, D):
    n_blocks = B // _LANE
    blocks_per_w = n_blocks // _NW
    n_bands = D // 8
    mesh = plsc.VectorSubcoreMesh(core_axis_name="c", subcore_axis_name="s")

    @functools.partial(
        pl.kernel,
        out_type=jax.ShapeDtypeStruct((n_bands, n_blocks, 8, _LANE), jnp.float32),
        mesh=mesh,
        scratch_types=[
            pltpu.VMEM((blocks_per_w, _LANE), jnp.int32),
            pltpu.VMEM((_LANE, D), jnp.float32),
            pltpu.VMEM((_LANE, D), jnp.float32),
            pltpu.VMEM((_LANE, D), jnp.float32),
            pltpu.VMEM((_LANE, D), jnp.float32),
            pltpu.VMEM((D, _LANE + 1), jnp.float32),
            pltpu.VMEM((D, _LANE + 1), jnp.float32),
            pltpu.SemaphoreType.DMA,
            pltpu.SemaphoreType.DMA,
            pltpu.SemaphoreType.DMA,
            pltpu.SemaphoreType.DMA,
            pltpu.SemaphoreType.DMA,
            pltpu.SemaphoreType.DMA,
        ],
        compiler_params=pltpu.CompilerParams(
            use_tc_tiling_on_sc=False, needs_layout_passes=False
        ),
    )
    def gather_kernel(idx_hbm, table_hbm, out4, idx_v,
                      rows_a, rows_b, rows_c, rows_d,
                      band_a, band_b,
                      gsem_a, gsem_b, gsem_c, gsem_d, osem_a, osem_b):
        rows = (rows_a, rows_b, rows_c, rows_d)
        bands = (band_a, band_b)
        gsems = (gsem_a, gsem_b, gsem_c, gsem_d)
        osems = (osem_a, osem_b)
        wid = lax.axis_index("s") * _NC + lax.axis_index("c")
        base = wid * blocks_per_w
        pltpu.sync_copy(idx_hbm.at[pl.ds(base, blocks_per_w)], idx_v)

        def fire(slot, i):
            pltpu.async_copy(table_hbm.at[idx_v.at[i]], rows[slot], gsems[slot])

        def wait_gather(slot):
            pltpu.make_async_copy(
                table_hbm.at[pl.ds(0, _LANE)], rows[slot], gsems[slot]
            ).wait()

        def write_bands(slot, nt):
            for g in range(n_bands):
                pltpu.async_copy(
                    bands[slot].at[pl.ds(8 * g, 8), pl.ds(0, _LANE)],
                    out4.at[g, nt],
                    osems[slot],
                )

        def wait_bands(slot):
            for g in range(n_bands):
                pltpu.make_async_copy(
                    bands[slot].at[pl.ds(8 * g, 8), pl.ds(0, _LANE)],
                    out4.at[g, 0],
                    osems[slot],
                ).wait()

        iotas = [16 * h + _iota16() for h in range(D // 16)]

        def transpose(rslot, bslot):
            src, dst = rows[rslot], bands[bslot]

            @plsc.parallel_loop(0, _LANE, unroll=4)
            def _(l):
                sl = _splat16(l)
                for h in range(D // 16):
                    v = src[l, pl.ds(16 * h, 16)]
                    plsc.store_scatter(dst, [iotas[h], sl], v)

        for s in range(3):
            fire(s, s)

        def step(t4, carry):
            for b in range(4):
                i = 4 * t4 + b

                @pl.when(i + 3 < blocks_per_w)
                def _():
                    fire((b + 3) % 4, i + 3)

                wait_gather(b)

                @pl.when(i >= 2)
                def _():
                    wait_bands(b % 2)

                transpose(b, b % 2)
                write_bands(b % 2, base + i)
            return carry

        lax.fori_loop(0, blocks_per_w // 4, step, 0)
        wait_bands(0)
        wait_bands(1)

    return gather_kernel


def kernel(walks, table):
    (B,) = walks.shape
    V, D = table.shape
    idx2d = walks.astype(jnp.int32).reshape(B // _LANE, _LANE)
    out4 = _make_gather(B, V, D)(idx2d, table)
    return out4.transpose(1, 3, 0, 2).reshape(B, D)


# speedup vs baseline: 1.7957x; 1.7316x over previous
import functools

import jax
import jax.numpy as jnp
from jax import lax
from jax.experimental import pallas as pl
from jax.experimental.pallas import tpu as pltpu
from jax.experimental.pallas import tpu_sc as plsc

_NC = 2
_NS = 16
_NW = _NC * _NS
_LANE = 128


def _iota16():
    return lax.iota(jnp.int32, 16)


def _splat16(v):
    return jnp.full((16,), v, jnp.int32)


@functools.lru_cache(maxsize=None)
def _make_detile(V, D):
    n_full = V // _LANE
    base_cnt, extra = divmod(n_full, _NW)
    mesh = plsc.VectorSubcoreMesh(core_axis_name="c", subcore_axis_name="s")

    @functools.partial(
        pl.kernel,
        out_type=jax.ShapeDtypeStruct((n_full, D, _LANE), jnp.float32),
        mesh=mesh,
        scratch_types=[
            pltpu.VMEM((D, _LANE), jnp.float32),
            pltpu.VMEM((D, _LANE), jnp.float32),
            pltpu.VMEM((D, _LANE), jnp.float32),
            pltpu.SemaphoreType.DMA,
            pltpu.SemaphoreType.DMA,
            pltpu.SemaphoreType.DMA,
            pltpu.SemaphoreType.DMA,
            pltpu.SemaphoreType.DMA,
            pltpu.SemaphoreType.DMA,
        ],
        compiler_params=pltpu.CompilerParams(use_tc_tiling_on_sc=True),
    )
    def detile_kernel(table_t, s_blocks, buf_a, buf_b, buf_c,
                      rsem_a, rsem_b, rsem_c, wsem_a, wsem_b, wsem_c):
        bufs = (buf_a, buf_b, buf_c)
        rsems = (rsem_a, rsem_b, rsem_c)
        wsems = (wsem_a, wsem_b, wsem_c)
        wid = lax.axis_index("s") * _NC + lax.axis_index("c")
        start = wid * base_cnt + jnp.minimum(wid, extra)
        count = base_cnt + jnp.where(wid < extra, 1, 0)

        def fire_read(slot, tc):
            pltpu.async_copy(
                table_t.at[:, pl.ds(pl.multiple_of(tc * _LANE, _LANE), _LANE)],
                bufs[slot], rsems[slot],
            )

        def wait_read(slot):
            pltpu.make_async_copy(
                table_t.at[:, pl.ds(0, _LANE)], bufs[slot], rsems[slot]
            ).wait()

        def fire_write(slot, tc):
            pltpu.async_copy(bufs[slot], s_blocks.at[tc], wsems[slot])

        def wait_write(slot):
            pltpu.make_async_copy(
                bufs[slot], s_blocks.at[0], wsems[slot]
            ).wait()

        fire_read(0, start)
        fire_read(1, start + 1)

        def step(t3, carry):
            for b in range(3):
                i = 3 * t3 + b

                @pl.when(i < count)
                def _():
                    @pl.when(i + 2 < count)
                    def _():
                        fire_read((b + 2) % 3, start + i + 2)

                    wait_read(b)

                    @pl.when(i >= 3)
                    def _():
                        wait_write(b)

                    fire_write(b, start + i)
            return carry

        lax.fori_loop(0, (base_cnt + 3) // 3, step, 0)
        wait_write(0)
        wait_write(1)
        wait_write(2)

    return detile_kernel


@functools.lru_cache(maxsize=None)
def _make_transpose(V, D):
    n_full = V // _LANE
    tail = V - n_full * _LANE
    rows_per_tile = _LANE * D // _LANE
    rows_out = V * D // _LANE
    tail_rows = tail * D // _LANE
    base_cnt, extra = divmod(n_full, _NW)
    mesh = plsc.VectorSubcoreMesh(core_axis_name="c", subcore_axis_name="s")

    @functools.partial(
        pl.kernel,
        out_type=jax.ShapeDtypeStruct((rows_out, _LANE), jnp.float32),
        mesh=mesh,
        scratch_types=[
            pltpu.VMEM((D, _LANE + 1), jnp.float32),
            pltpu.VMEM((D, _LANE + 1), jnp.float32),
            pltpu.VMEM((D, _LANE + 1), jnp.float32),
            pltpu.VMEM((rows_per_tile, _LANE), jnp.float32),
            pltpu.VMEM((rows_per_tile, _LANE), jnp.float32),
            pltpu.VMEM((tail, D), jnp.float32),
            pltpu.SemaphoreType.DMA,
            pltpu.SemaphoreType.DMA,
            pltpu.SemaphoreType.DMA,
            pltpu.SemaphoreType.DMA,
            pltpu.SemaphoreType.DMA,
        ],
        compiler_params=pltpu.CompilerParams(
            use_tc_tiling_on_sc=False, needs_layout_passes=False
        ),
    )
    def transpose_kernel(s_blocks, tail_in, table_lin, f_a, f_b, f_c, r_a, r_b,
                         tbuf, rsem_a, rsem_b, rsem_c, wsem_a, wsem_b):
        fs = (f_a, f_b, f_c)
        rs = (r_a, r_b)
        rsems = (rsem_a, rsem_b, rsem_c)
        wsems = (wsem_a, wsem_b)
        wid = lax.axis_index("s") * _NC + lax.axis_index("c")
        start = wid * base_cnt + jnp.minimum(wid, extra)
        count = base_cnt + jnp.where(wid < extra, 1, 0)

        def fire_read(slot, tc):
            pltpu.async_copy(
                s_blocks.at[tc], fs[slot].at[:, pl.ds(0, _LANE)], rsems[slot]
            )

        def wait_read(slot):
            pltpu.make_async_copy(
                s_blocks.at[0], fs[slot].at[:, pl.ds(0, _LANE)], rsems[slot]
            ).wait()

        def fire_write(slot, tc):
            pltpu.async_copy(
                rs[slot],
                table_lin.at[pl.ds(tc * rows_per_tile, rows_per_tile)],
                wsems[slot],
            )

        def wait_write(slot):
            pltpu.make_async_copy(
                rs[slot], table_lin.at[pl.ds(0, rows_per_tile)], wsems[slot]
            ).wait()

        iotas = [16 * h + _iota16() for h in range(D // 16)]

        def transpose(fslot, rslot):
            src, dst = fs[fslot], rs[rslot]

            @plsc.parallel_loop(0, rows_per_tile, unroll=4)
            def _(q):
                for c in range(8):
                    a, f0 = c // 2, (c % 2) * 16
                    v = plsc.load_gather(src, [iotas[c % 2], _splat16(4 * q + a)])
                    dst[q, pl.ds(16 * c, 16)] = v

        fire_read(0, start)
        fire_read(1, start + 1)

        def step(t6, carry):
            for b6 in range(6):
                i = 6 * t6 + b6
                b3, b2 = b6 % 3, b6 % 2

                @pl.when(i < count)
                def _():
                    @pl.when(i + 2 < count)
                    def _():
                        fire_read((b3 + 2) % 3, start + i + 2)

                    wait_read(b3)

                    @pl.when(i >= 2)
                    def _():
                        wait_write(b2)

                    transpose(b3, b2)
                    fire_write(b2, start + i)
            return carry

        lax.fori_loop(0, (base_cnt + 6) // 6, step, 0)
        wait_write(0)
        wait_write(1)

        @pl.when(wid == _NW - 1)
        def _():
            pltpu.sync_copy(tail_in, tbuf)

            @plsc.parallel_loop(0, tail_rows, unroll=2)
            def _(r):
                for h in range(_LANE // 16):
                    v = tbuf[4 * r + h // 2, pl.ds((h % 2) * 16, 16)]
                    r_a[r, pl.ds(16 * h, 16)] = v

            pltpu.sync_copy(
                r_a.at[pl.ds(0, tail_rows)],
                table_lin.at[pl.ds(n_full * rows_per_tile, tail_rows)],
            )

    return transpose_kernel


@functools.lru_cache(maxsize=None)
def _make_gather(B, V, D):
    n_blocks = B // _LANE
    blocks_per_w = n_blocks // _NW
    n_bands = D // 8
    mesh = plsc.VectorSubcoreMesh(core_axis_name="c", subcore_axis_name="s")

    @functools.partial(
        pl.kernel,
        out_type=jax.ShapeDtypeStruct((n_bands, n_blocks, 8, _LANE), jnp.float32),
        mesh=mesh,
        scratch_types=[
            pltpu.VMEM((blocks_per_w, _LANE), jnp.int32),
            pltpu.VMEM((_LANE, D), jnp.float32),
            pltpu.VMEM((_LANE, D), jnp.float32),
            pltpu.VMEM((_LANE, D), jnp.float32),
            pltpu.VMEM((_LANE, D), jnp.float32),
            pltpu.VMEM((D, _LANE + 1), jnp.float32),
            pltpu.VMEM((D, _LANE + 1), jnp.float32),
            pltpu.SemaphoreType.DMA,
            pltpu.SemaphoreType.DMA,
            pltpu.SemaphoreType.DMA,
            pltpu.SemaphoreType.DMA,
            pltpu.SemaphoreType.DMA,
            pltpu.SemaphoreType.DMA,
        ],
        compiler_params=pltpu.CompilerParams(
            use_tc_tiling_on_sc=False, needs_layout_passes=False
        ),
    )
    def gather_kernel(idx_hbm, table_hbm, out4, idx_v,
                      rows_a, rows_b, rows_c, rows_d,
                      band_a, band_b,
                      gsem_a, gsem_b, gsem_c, gsem_d, osem_a, osem_b):
        rows = (rows_a, rows_b, rows_c, rows_d)
        bands = (band_a, band_b)
        gsems = (gsem_a, gsem_b, gsem_c, gsem_d)
        osems = (osem_a, osem_b)
        wid = lax.axis_index("s") * _NC + lax.axis_index("c")
        base = wid * blocks_per_w
        pltpu.sync_copy(idx_hbm.at[pl.ds(base, blocks_per_w)], idx_v)

        def fire(slot, i):
            pltpu.async_copy(table_hbm.at[idx_v.at[i]], rows[slot], gsems[slot])

        def wait_gather(slot):
            pltpu.make_async_copy(
                table_hbm.at[pl.ds(0, _LANE)], rows[slot], gsems[slot]
            ).wait()

        def write_bands(slot, nt):
            for g in range(n_bands):
                pltpu.async_copy(
                    bands[slot].at[pl.ds(8 * g, 8), pl.ds(0, _LANE)],
                    out4.at[g, nt],
                    osems[slot],
                )

        def wait_bands(slot):
            for g in range(n_bands):
                pltpu.make_async_copy(
                    bands[slot].at[pl.ds(8 * g, 8), pl.ds(0, _LANE)],
                    out4.at[g, 0],
                    osems[slot],
                ).wait()

        iotas = [16 * h + _iota16() for h in range(D // 16)]

        def transpose(rslot, bslot):
            src, dst = rows[rslot], bands[bslot]

            @plsc.parallel_loop(0, _LANE, unroll=4)
            def _(l):
                sl = _splat16(l)
                for h in range(D // 16):
                    v = src[l, pl.ds(16 * h, 16)]
                    plsc.store_scatter(dst, [iotas[h], sl], v)

        for s in range(3):
            fire(s, s)

        def step(t4, carry):
            for b in range(4):
                i = 4 * t4 + b

                @pl.when(i + 3 < blocks_per_w)
                def _():
                    fire((b + 3) % 4, i + 3)

                wait_gather(b)

                @pl.when(i >= 2)
                def _():
                    wait_bands(b % 2)

                transpose(b, b % 2)
                write_bands(b % 2, base + i)
            return carry

        lax.fori_loop(0, blocks_per_w // 4, step, 0)
        wait_bands(0)
        wait_bands(1)

    return gather_kernel


def kernel(walks, table):
    (B,) = walks.shape
    V, D = table.shape
    n_full = V // _LANE
    s_blocks = _make_detile(V, D)(jnp.transpose(table))
    tail64 = lax.slice(table, (n_full * _LANE, 0), (V, D))
    table_lin = _make_transpose(V, D)(s_blocks, tail64)
    table_rows = table_lin.reshape(V, D)
    idx2d = walks.astype(jnp.int32).reshape(B // _LANE, _LANE)
    out4 = _make_gather(B, V, D)(idx2d, table_rows)
    return out4.transpose(1, 3, 0, 2).reshape(B, D)
